# baseline (speedup 1.0000x reference)
import functools

import jax
import jax.numpy as jnp
from jax import lax
from jax.experimental import pallas as pl
from jax.experimental.pallas import tpu as pltpu
from jax.experimental.pallas import tpu_sc as plsc

N_NODES = 10000
N_EDGES = 320000
D_FEAT = 128

NUM_CORES = 2
NUM_SUBCORES = 16
LANES = 16
NUM_WORKERS = NUM_CORES * NUM_SUBCORES
EDGES_MAIN = 9984
TAIL_START = NUM_WORKERS * EDGES_MAIN
TAIL_CHUNK = 128


N_NODES_PAD = 10240
ROW_BLK = 1280


def _node_table_body(h_ref, w1_ref, b1_ref, w2_ref, b2_ref, tab_ref, m_s, bias_s):
    @pl.when(pl.program_id(0) == 0)
    def _():
        v = jnp.dot(w1_ref[...], w2_ref[...], preferred_element_type=jnp.float32)
        m_s[...] = jnp.concatenate([v[:D_FEAT, :], v[D_FEAT:, :]], axis=1)
        cb = jnp.dot(b1_ref[...], w2_ref[...], preferred_element_type=jnp.float32)
        cb = cb + b2_ref[...]
        bias_s[...] = jnp.concatenate([cb, jnp.zeros((1, 1), jnp.float32)], axis=1)

    r = jnp.dot(h_ref[...], m_s[...], preferred_element_type=jnp.float32)
    tab_ref[...] = (r + bias_s[...]).T.reshape(2, 1, ROW_BLK)


N_BLOCKS = 3
BLK = EDGES_MAIN // N_BLOCKS


def _edge_sum_body(
    tab_hbm,
    edges_hbm,
    out_hbm,
    a_v,
    b_v,
    e0_v,
    e1_v,
    t_v,
    o0_v,
    o1_v,
    outt_v,
    sem_a,
    sem_b,
    sem_e0,
    sem_e1,
    sem_o0,
    sem_o1,
    sem_x,
):
    wid = lax.axis_index("s") * NUM_CORES + lax.axis_index("c")
    base = pl.multiple_of(wid * EDGES_MAIN, 128)
    tail_base = pl.multiple_of(TAIL_START + wid * TAIL_CHUNK, 128)
    has_tail = wid < NUM_CORES * 2

    e_bufs = (e0_v, e1_v)
    e_sems = (sem_e0, sem_e1)
    o_bufs = (o0_v, o1_v)
    o_sems = (sem_o0, sem_o1)

    def stage(i):
        slot = i % 2
        pltpu.make_async_copy(
            edges_hbm.at[pl.ds(0, 2), pl.ds(base + i * BLK, BLK)],
            e_bufs[slot],
            e_sems[slot],
        ).start()

    cp_a = pltpu.make_async_copy(tab_hbm.at[0, 0], a_v, sem_a)
    cp_b = pltpu.make_async_copy(tab_hbm.at[1, 0], b_v, sem_b)
    cp_a.start()
    cp_b.start()
    stage(0)
    stage(1)
    cp_x = pltpu.make_async_copy(
        edges_hbm.at[pl.ds(0, 2), pl.ds(tail_base, TAIL_CHUNK)], t_v, sem_x
    )

    @pl.when(has_tail)
    def _():
        cp_x.start()

    cp_a.wait()
    cp_b.wait()

    for i in range(N_BLOCKS):
        slot = i % 2
        e_v = e_bufs[slot]
        o_v = o_bufs[slot]
        pltpu.make_async_copy(
            edges_hbm.at[pl.ds(0, 2), pl.ds(base + i * BLK, BLK)],
            e_v,
            e_sems[slot],
        ).wait()
        if i >= 2:
            pltpu.make_async_copy(
                o_v, out_hbm.at[0, pl.ds(base + (i - 2) * BLK, BLK)], o_sems[slot]
            ).wait()
        if i + 1 < N_BLOCKS:
            stage(i + 1)

        @plsc.parallel_loop(0, BLK, LANES, unroll=8)
        def step(off):
            si = e_v[0, pl.ds(off, LANES)]
            di = e_v[1, pl.ds(off, LANES)]
            av = plsc.load_gather(a_v, [si])
            bv = plsc.load_gather(b_v, [di])
            o_v[pl.ds(off, LANES)] = av + bv

        pltpu.make_async_copy(
            o_v, out_hbm.at[0, pl.ds(base + i * BLK, BLK)], o_sems[slot]
        ).start()

    @pl.when(has_tail)
    def _():
        cp_x.wait()

        @plsc.parallel_loop(0, TAIL_CHUNK, LANES, unroll=8)
        def tail_step(off):
            si = t_v[0, pl.ds(off, LANES)]
            di = t_v[1, pl.ds(off, LANES)]
            av = plsc.load_gather(a_v, [si])
            bv = plsc.load_gather(b_v, [di])
            outt_v[pl.ds(off, LANES)] = av + bv

        pltpu.sync_copy(outt_v, out_hbm.at[0, pl.ds(tail_base, TAIL_CHUNK)])

    for i in (N_BLOCKS - 2, N_BLOCKS - 1):
        slot = i % 2
        pltpu.make_async_copy(
            o_bufs[slot], out_hbm.at[0, pl.ds(base + i * BLK, BLK)], o_sems[slot]
        ).wait()


def kernel(hiddens, edges, W1, b1, W2, b2):
    tab = pl.pallas_call(
        _node_table_body,
        grid=(N_NODES_PAD // ROW_BLK,),
        in_specs=[
            pl.BlockSpec((ROW_BLK, D_FEAT), lambda i: (i, 0)),
            pl.BlockSpec((2 * D_FEAT, 16), lambda i: (0, 0)),
            pl.BlockSpec((1, 16), lambda i: (0, 0)),
            pl.BlockSpec((16, 1), lambda i: (0, 0)),
            pl.BlockSpec((1, 1), lambda i: (0, 0)),
        ],
        out_specs=pl.BlockSpec((2, 1, ROW_BLK), lambda i: (0, 0, i)),
        out_shape=jax.ShapeDtypeStruct((2, 1, N_NODES_PAD), jnp.float32),
        scratch_shapes=[
            pltpu.VMEM((D_FEAT, 2), jnp.float32),
            pltpu.VMEM((1, 2), jnp.float32),
        ],
    )(
        hiddens,
        W1,
        b1.reshape(1, 16),
        W2,
        b2.reshape(1, 1),
    )

    mesh = plsc.VectorSubcoreMesh(core_axis_name="c", subcore_axis_name="s")
    edge_sum = functools.partial(
        pl.kernel,
        out_type=jax.ShapeDtypeStruct((1, N_EDGES), jnp.float32),
        mesh=mesh,
        compiler_params=pltpu.CompilerParams(needs_layout_passes=False),
        scratch_types=[
            pltpu.VMEM((N_NODES_PAD,), jnp.float32),
            pltpu.VMEM((N_NODES_PAD,), jnp.float32),
            pltpu.VMEM((2, BLK), jnp.int32),
            pltpu.VMEM((2, BLK), jnp.int32),
            pltpu.VMEM((2, TAIL_CHUNK), jnp.int32),
            pltpu.VMEM((BLK,), jnp.float32),
            pltpu.VMEM((BLK,), jnp.float32),
            pltpu.VMEM((TAIL_CHUNK,), jnp.float32),
            pltpu.SemaphoreType.DMA,
            pltpu.SemaphoreType.DMA,
            pltpu.SemaphoreType.DMA,
            pltpu.SemaphoreType.DMA,
            pltpu.SemaphoreType.DMA,
            pltpu.SemaphoreType.DMA,
            pltpu.SemaphoreType.DMA,
        ],
    )(_edge_sum_body)

    logits = edge_sum(tab, edges.astype(jnp.int32))
    return logits.reshape(N_EDGES, 1)


# speedup vs baseline: 1.0749x; 1.0749x over previous
import functools

import jax
import jax.numpy as jnp
from jax import lax
from jax.experimental import pallas as pl
from jax.experimental.pallas import tpu as pltpu
from jax.experimental.pallas import tpu_sc as plsc

N_NODES = 10000
N_EDGES = 320000
D_FEAT = 128

NUM_CORES = 2
NUM_SUBCORES = 16
LANES = 16
NUM_WORKERS = NUM_CORES * NUM_SUBCORES
EDGES_MAIN = 9984
TAIL_START = NUM_WORKERS * EDGES_MAIN
TAIL_CHUNK = 128


def _node_table_body(h_ref, w1_ref, b1_ref, w2_ref, b2_ref, tab_ref):
    v = jnp.dot(w1_ref[...], w2_ref[...], preferred_element_type=jnp.float32)
    m = jnp.concatenate([v[:D_FEAT, :], v[D_FEAT:, :]], axis=1)
    cb = jnp.dot(b1_ref[...], w2_ref[...], preferred_element_type=jnp.float32)
    cb = cb + b2_ref[...]
    bias_row = jnp.concatenate([cb, jnp.zeros((1, 1), jnp.float32)], axis=1)
    r = jnp.dot(h_ref[...], m, preferred_element_type=jnp.float32) + bias_row
    tab_ref[...] = r.T.reshape(2, 1, N_NODES)


N_BLOCKS = 3
BLK = EDGES_MAIN // N_BLOCKS


def _edge_sum_body(
    tab_hbm,
    edges_hbm,
    out_hbm,
    a_v,
    b_v,
    e0_v,
    e1_v,
    t_v,
    o0_v,
    o1_v,
    outt_v,
    sem_a,
    sem_b,
    sem_e0,
    sem_e1,
    sem_o0,
    sem_o1,
    sem_x,
):
    wid = lax.axis_index("s") * NUM_CORES + lax.axis_index("c")
    base = pl.multiple_of(wid * EDGES_MAIN, 128)
    tail_base = pl.multiple_of(TAIL_START + wid * TAIL_CHUNK, 128)
    has_tail = wid < NUM_CORES * 2

    e_bufs = (e0_v, e1_v)
    e_sems = (sem_e0, sem_e1)
    o_bufs = (o0_v, o1_v)
    o_sems = (sem_o0, sem_o1)

    def stage(i):
        slot = i % 2
        pltpu.make_async_copy(
            edges_hbm.at[pl.ds(0, 2), pl.ds(base + i * BLK, BLK)],
            e_bufs[slot],
            e_sems[slot],
        ).start()

    cp_a = pltpu.make_async_copy(tab_hbm.at[0, 0], a_v, sem_a)
    cp_b = pltpu.make_async_copy(tab_hbm.at[1, 0], b_v, sem_b)
    cp_a.start()
    cp_b.start()
    stage(0)
    stage(1)
    cp_x = pltpu.make_async_copy(
        edges_hbm.at[pl.ds(0, 2), pl.ds(tail_base, TAIL_CHUNK)], t_v, sem_x
    )

    @pl.when(has_tail)
    def _():
        cp_x.start()

    cp_a.wait()
    cp_b.wait()

    for i in range(N_BLOCKS):
        slot = i % 2
        e_v = e_bufs[slot]
        o_v = o_bufs[slot]
        pltpu.make_async_copy(
            edges_hbm.at[pl.ds(0, 2), pl.ds(base + i * BLK, BLK)],
            e_v,
            e_sems[slot],
        ).wait()
        if i >= 2:
            pltpu.make_async_copy(
                o_v, out_hbm.at[0, pl.ds(base + (i - 2) * BLK, BLK)], o_sems[slot]
            ).wait()
        if i + 1 < N_BLOCKS:
            stage(i + 1)

        @plsc.parallel_loop(0, BLK, LANES, unroll=16)
        def step(off):
            si = e_v[0, pl.ds(off, LANES)]
            di = e_v[1, pl.ds(off, LANES)]
            av = plsc.load_gather(a_v, [si])
            bv = plsc.load_gather(b_v, [di])
            o_v[pl.ds(off, LANES)] = av + bv

        pltpu.make_async_copy(
            o_v, out_hbm.at[0, pl.ds(base + i * BLK, BLK)], o_sems[slot]
        ).start()

    @pl.when(has_tail)
    def _():
        cp_x.wait()

        @plsc.parallel_loop(0, TAIL_CHUNK, LANES, unroll=8)
        def tail_step(off):
            si = t_v[0, pl.ds(off, LANES)]
            di = t_v[1, pl.ds(off, LANES)]
            av = plsc.load_gather(a_v, [si])
            bv = plsc.load_gather(b_v, [di])
            outt_v[pl.ds(off, LANES)] = av + bv

        pltpu.sync_copy(outt_v, out_hbm.at[0, pl.ds(tail_base, TAIL_CHUNK)])

    for i in (N_BLOCKS - 2, N_BLOCKS - 1):
        slot = i % 2
        pltpu.make_async_copy(
            o_bufs[slot], out_hbm.at[0, pl.ds(base + i * BLK, BLK)], o_sems[slot]
        ).wait()


def kernel(hiddens, edges, W1, b1, W2, b2):
    tab = pl.pallas_call(
        _node_table_body,
        out_shape=jax.ShapeDtypeStruct((2, 1, N_NODES), jnp.float32),
    )(
        hiddens,
        W1,
        b1.reshape(1, 16),
        W2,
        b2.reshape(1, 1),
    )

    mesh = plsc.VectorSubcoreMesh(core_axis_name="c", subcore_axis_name="s")
    edge_sum = functools.partial(
        pl.kernel,
        out_type=jax.ShapeDtypeStruct((1, N_EDGES), jnp.float32),
        mesh=mesh,
        compiler_params=pltpu.CompilerParams(needs_layout_passes=False),
        scratch_types=[
            pltpu.VMEM((N_NODES,), jnp.float32),
            pltpu.VMEM((N_NODES,), jnp.float32),
            pltpu.VMEM((2, BLK), jnp.int32),
            pltpu.VMEM((2, BLK), jnp.int32),
            pltpu.VMEM((2, TAIL_CHUNK), jnp.int32),
            pltpu.VMEM((BLK,), jnp.float32),
            pltpu.VMEM((BLK,), jnp.float32),
            pltpu.VMEM((TAIL_CHUNK,), jnp.float32),
            pltpu.SemaphoreType.DMA,
            pltpu.SemaphoreType.DMA,
            pltpu.SemaphoreType.DMA,
            pltpu.SemaphoreType.DMA,
            pltpu.SemaphoreType.DMA,
            pltpu.SemaphoreType.DMA,
            pltpu.SemaphoreType.DMA,
        ],
    )(_edge_sum_body)

    logits = edge_sum(tab, edges.astype(jnp.int32))
    return logits.reshape(N_EDGES, 1)


# speedup vs baseline: 1.1003x; 1.0236x over previous
import functools

import jax
import jax.numpy as jnp
from jax import lax
from jax.experimental import pallas as pl
from jax.experimental.pallas import tpu as pltpu
from jax.experimental.pallas import tpu_sc as plsc

N_NODES = 10000
N_EDGES = 320000
D_FEAT = 128

NUM_CORES = 2
NUM_SUBCORES = 16
LANES = 16
NUM_WORKERS = NUM_CORES * NUM_SUBCORES
EDGES_MAIN = 9984
TAIL_START = NUM_WORKERS * EDGES_MAIN
TAIL_CHUNK = 128


def _node_table_body(h_ref, w1_ref, b1_ref, w2_ref, b2_ref, tab_ref):
    v = jnp.dot(w1_ref[...], w2_ref[...], preferred_element_type=jnp.float32)
    m = jnp.concatenate([v[:D_FEAT, :], v[D_FEAT:, :]], axis=1)
    cb = jnp.dot(b1_ref[...], w2_ref[...], preferred_element_type=jnp.float32)
    cb = cb + b2_ref[...]
    bias_row = jnp.concatenate([cb, jnp.zeros((1, 1), jnp.float32)], axis=1)
    r = jnp.dot(h_ref[...], m, preferred_element_type=jnp.float32) + bias_row
    tab_ref[...] = r.T.reshape(2, 1, N_NODES)


N_BLOCKS = 3
BLK = EDGES_MAIN // N_BLOCKS


def _edge_sum_body(
    tab_hbm,
    edges_hbm,
    out_hbm,
    a_v,
    b_v,
    e0_v,
    e1_v,
    t_v,
    o0_v,
    o1_v,
    outt_v,
    sem_a,
    sem_b,
    sem_e0,
    sem_e1,
    sem_o0,
    sem_o1,
    sem_x,
):
    wid = lax.axis_index("s") * NUM_CORES + lax.axis_index("c")
    base = pl.multiple_of(wid * EDGES_MAIN, 128)
    tail_base = pl.multiple_of(TAIL_START + wid * TAIL_CHUNK, 128)
    has_tail = wid < NUM_CORES * 2

    e_bufs = (e0_v, e1_v)
    e_sems = (sem_e0, sem_e1)
    o_bufs = (o0_v, o1_v)
    o_sems = (sem_o0, sem_o1)

    def stage(i):
        slot = i % 2
        pltpu.make_async_copy(
            edges_hbm.at[pl.ds(0, 2), pl.ds(base + i * BLK, BLK)],
            e_bufs[slot],
            e_sems[slot],
        ).start()

    cp_a = pltpu.make_async_copy(tab_hbm.at[0, 0], a_v, sem_a)
    cp_b = pltpu.make_async_copy(tab_hbm.at[1, 0], b_v, sem_b)
    cp_a.start()
    cp_b.start()
    stage(0)
    stage(1)
    cp_x = pltpu.make_async_copy(
        edges_hbm.at[pl.ds(0, 2), pl.ds(tail_base, TAIL_CHUNK)], t_v, sem_x
    )

    @pl.when(has_tail)
    def _():
        cp_x.start()

    cp_a.wait()
    cp_b.wait()

    for i in range(N_BLOCKS):
        slot = i % 2
        e_v = e_bufs[slot]
        o_v = o_bufs[slot]
        pltpu.make_async_copy(
            edges_hbm.at[pl.ds(0, 2), pl.ds(base + i * BLK, BLK)],
            e_v,
            e_sems[slot],
        ).wait()
        if i >= 2:
            pltpu.make_async_copy(
                o_v, out_hbm.at[0, pl.ds(base + (i - 2) * BLK, BLK)], o_sems[slot]
            ).wait()
        if i + 1 < N_BLOCKS:
            stage(i + 1)

        @plsc.parallel_loop(0, BLK, LANES, unroll=8)
        def step(off):
            si = e_v[0, pl.ds(off, LANES)]
            di = e_v[1, pl.ds(off, LANES)]
            av = plsc.load_gather(a_v, [si])
            bv = plsc.load_gather(b_v, [di])
            o_v[pl.ds(off, LANES)] = av + bv

        pltpu.make_async_copy(
            o_v, out_hbm.at[0, pl.ds(base + i * BLK, BLK)], o_sems[slot]
        ).start()

    @pl.when(has_tail)
    def _():
        cp_x.wait()

        @plsc.parallel_loop(0, TAIL_CHUNK, LANES, unroll=8)
        def tail_step(off):
            si = t_v[0, pl.ds(off, LANES)]
            di = t_v[1, pl.ds(off, LANES)]
            av = plsc.load_gather(a_v, [si])
            bv = plsc.load_gather(b_v, [di])
            outt_v[pl.ds(off, LANES)] = av + bv

        pltpu.sync_copy(outt_v, out_hbm.at[0, pl.ds(tail_base, TAIL_CHUNK)])

    for i in (N_BLOCKS - 2, N_BLOCKS - 1):
        slot = i % 2
        pltpu.make_async_copy(
            o_bufs[slot], out_hbm.at[0, pl.ds(base + i * BLK, BLK)], o_sems[slot]
        ).wait()


def kernel(hiddens, edges, W1, b1, W2, b2):
    tab = pl.pallas_call(
        _node_table_body,
        out_shape=jax.ShapeDtypeStruct((2, 1, N_NODES), jnp.float32),
    )(
        hiddens,
        W1,
        b1.reshape(1, 16),
        W2,
        b2.reshape(1, 1),
    )

    mesh = plsc.VectorSubcoreMesh(core_axis_name="c", subcore_axis_name="s")
    edge_sum = functools.partial(
        pl.kernel,
        out_type=jax.ShapeDtypeStruct((1, N_EDGES), jnp.float32),
        mesh=mesh,
        compiler_params=pltpu.CompilerParams(needs_layout_passes=False),
        scratch_types=[
            pltpu.VMEM((N_NODES,), jnp.float32),
            pltpu.VMEM((N_NODES,), jnp.float32),
            pltpu.VMEM((2, BLK), jnp.int32),
            pltpu.VMEM((2, BLK), jnp.int32),
            pltpu.VMEM((2, TAIL_CHUNK), jnp.int32),
            pltpu.VMEM((BLK,), jnp.float32),
            pltpu.VMEM((BLK,), jnp.float32),
            pltpu.VMEM((TAIL_CHUNK,), jnp.float32),
            pltpu.SemaphoreType.DMA,
            pltpu.SemaphoreType.DMA,
            pltpu.SemaphoreType.DMA,
            pltpu.SemaphoreType.DMA,
            pltpu.SemaphoreType.DMA,
            pltpu.SemaphoreType.DMA,
            pltpu.SemaphoreType.DMA,
        ],
    )(_edge_sum_body)

    logits = edge_sum(tab, edges.astype(jnp.int32))
    return logits.reshape(N_EDGES, 1)
